# Initial kernel scaffold; baseline (speedup 1.0000x reference)
import jax
import jax.numpy as jnp
from jax.experimental import pallas as pl


def kernel(x, edge_index, edge_weight, W1, b1, W2, b2, W3, b3):
    raise NotImplementedError("write your pallas kernel here")


# speedup vs baseline: 7.9713x; 7.9713x over previous
import functools

import jax
import jax.numpy as jnp
from jax import lax
from jax.experimental import pallas as pl
from jax.experimental.pallas import tpu as pltpu
from jax.experimental.pallas import tpu_sc as plsc

NC = 2
NS = 16
NW = NC * NS
B = 128
LANES = 16


@functools.lru_cache(maxsize=None)
def _build_deg_kernel(n, pt):
    mesh = plsc.VectorSubcoreMesh(core_axis_name="c", subcore_axis_name="s",
                                  num_cores=NC, num_subcores=NS)

    @functools.partial(
        pl.kernel,
        out_type=jax.ShapeDtypeStruct((NW, 1, n), jnp.float32),
        mesh=mesh,
        scratch_types=[
            pltpu.VMEM((pt,), jnp.int32),
            pltpu.VMEM((pt,), jnp.float32),
            pltpu.VMEM((n,), jnp.float32),
        ],
        compiler_params=pltpu.CompilerParams(needs_layout_passes=False),
    )
    def deg_kernel(dst_hbm, ew_hbm, out_hbm, dst_v, ew_v, deg_v):
        cid = lax.axis_index("c")
        sid = lax.axis_index("s")
        wid = cid * NS + sid

        pltpu.sync_copy(dst_hbm.at[pl.ds(wid * pt, pt)], dst_v)
        pltpu.sync_copy(ew_hbm.at[pl.ds(wid * pt, pt)], ew_v)

        def zero_body(i, _):
            deg_v[pl.ds(i * LANES, LANES)] = jnp.zeros((LANES,), jnp.float32)
            return _
        lax.fori_loop(0, n // LANES, zero_body, None)

        def scat_body(i, _):
            idx = dst_v[pl.ds(i * LANES, LANES)]
            w = ew_v[pl.ds(i * LANES, LANES)]
            plsc.addupdate_scatter(deg_v, [idx], w)
            return _
        lax.fori_loop(0, pt // LANES, scat_body, None)

        pltpu.sync_copy(deg_v, out_hbm.at[wid, 0])

    return deg_kernel


@functools.lru_cache(maxsize=None)
def _build_spmm_kernel(n_pad, d, nb):
    mesh = plsc.VectorSubcoreMesh(core_axis_name="c", subcore_axis_name="s",
                                  num_cores=NC, num_subcores=NS)
    rows_per_tile = n_pad // NS
    n_full = rows_per_tile // B
    n_rem = rows_per_tile - n_full * B

    @functools.partial(
        pl.kernel,
        out_type=jax.ShapeDtypeStruct((NC, n_pad, d), jnp.float32),
        mesh=mesh,
        scratch_types=[
            pltpu.VMEM((B,), jnp.int32),
            pltpu.VMEM((B,), jnp.int32),
            pltpu.VMEM((B,), jnp.float32),
            pltpu.VMEM((B, d), jnp.float32),
            pltpu.VMEM_SHARED((n_pad, d), jnp.float32),
        ],
        compiler_params=pltpu.CompilerParams(needs_layout_passes=False),
    )
    def spmm_kernel(src_hbm, dst_hbm, ew_hbm, xs_hbm, out_hbm,
                    srcb_v, dstb_v, ewb_v, rows_v, acc_sh):
        cid = lax.axis_index("c")
        sid = lax.axis_index("s")
        wid = cid * NS + sid

        def zrow(e, _):
            for c in range(d // LANES):
                rows_v[e, pl.ds(c * LANES, LANES)] = (
                    jnp.zeros((LANES,), jnp.float32))
            return _
        lax.fori_loop(0, B, zrow, None)
        base = sid * rows_per_tile
        for k in range(n_full):
            pltpu.sync_copy(rows_v, acc_sh.at[pl.ds(base + k * B, B)])
        if n_rem:
            pltpu.sync_copy(rows_v.at[pl.ds(0, n_rem)],
                            acc_sh.at[pl.ds(base + n_full * B, n_rem)])
        plsc.subcore_barrier()

        def batch_body(j, _):
            row = wid * nb + j
            pltpu.sync_copy(src_hbm.at[row], srcb_v)
            pltpu.sync_copy(dst_hbm.at[row], dstb_v)
            pltpu.sync_copy(ew_hbm.at[row], ewb_v)
            pltpu.sync_copy(xs_hbm.at[srcb_v], rows_v)

            def scale(g, _):
                ewv = ewb_v[pl.ds(g * LANES, LANES)]
                for l in range(LANES):
                    s = ewv[l]
                    e = g * LANES + l
                    for c in range(d // LANES):
                        sl = pl.ds(c * LANES, LANES)
                        rows_v[e, sl] = rows_v[e, sl] * s
                return _
            lax.fori_loop(0, B // LANES, scale, None)

            pltpu.sync_copy(rows_v, acc_sh.at[dstb_v], add=True)
            return _
        lax.fori_loop(0, nb, batch_body, None)

        plsc.subcore_barrier()
        pltpu.sync_copy(acc_sh.at[pl.ds(base, rows_per_tile)],
                        out_hbm.at[cid, pl.ds(base, rows_per_tile)])

    return spmm_kernel


def _dis_body(degp_ref, dis_ref):
    deg = jnp.sum(degp_ref[...], axis=0) + 1.0
    dis = jnp.where(deg > 0, lax.rsqrt(deg), 0.0)
    dis_ref[...] = dis[:, None]


@functools.lru_cache(maxsize=None)
def _build_dis(n):
    return pl.pallas_call(
        _dis_body,
        out_shape=jax.ShapeDtypeStruct((n, 1), jnp.float32),
    )


def _xs_body(x_ref, dis_ref, w_ref, xs_ref):
    xs_ref[...] = dis_ref[...] * jnp.dot(
        x_ref[...], w_ref[...], preferred_element_type=jnp.float32)


@functools.lru_cache(maxsize=None)
def _build_xs(n, d, bn):
    grid = n // bn
    return pl.pallas_call(
        _xs_body,
        grid=(grid,),
        in_specs=[
            pl.BlockSpec((bn, d), lambda i: (i, 0)),
            pl.BlockSpec((bn, 1), lambda i: (i, 0)),
            pl.BlockSpec((d, d), lambda i: (0, 0)),
        ],
        out_specs=pl.BlockSpec((bn, d), lambda i: (i, 0)),
        out_shape=jax.ShapeDtypeStruct((n, d), jnp.float32),
    )


def _combine_body(acc_ref, xs_ref, dis_ref, b_ref, w_ref, out_ref):
    t = acc_ref[0] + acc_ref[1] + xs_ref[...]
    pre = dis_ref[...] * t + b_ref[...]
    h = pre * jax.nn.sigmoid(pre)
    out_ref[...] = dis_ref[...] * jnp.dot(
        h, w_ref[...], preferred_element_type=jnp.float32)


@functools.lru_cache(maxsize=None)
def _build_combine(n, d, bn):
    grid = n // bn
    return pl.pallas_call(
        _combine_body,
        grid=(grid,),
        in_specs=[
            pl.BlockSpec((NC, bn, d), lambda i: (0, i, 0)),
            pl.BlockSpec((bn, d), lambda i: (i, 0)),
            pl.BlockSpec((bn, 1), lambda i: (i, 0)),
            pl.BlockSpec((1, d), lambda i: (0, 0)),
            pl.BlockSpec((d, d), lambda i: (0, 0)),
        ],
        out_specs=pl.BlockSpec((bn, d), lambda i: (i, 0)),
        out_shape=jax.ShapeDtypeStruct((n, d), jnp.float32),
    )


@functools.lru_cache(maxsize=None)
def _build_final(n, d, bn):
    grid = n // bn
    inv_n = 1.0 / n

    def body(acc_ref, xs_ref, dis_ref, b_ref, out_ref):
        i = pl.program_id(0)
        t = acc_ref[0] + acc_ref[1] + xs_ref[...]
        pre = dis_ref[...] * t + b_ref[...]
        h = pre * jax.nn.sigmoid(pre)
        part = jnp.sum(h, axis=0, keepdims=True) * inv_n

        @pl.when(i == 0)
        def _():
            out_ref[...] = part

        @pl.when(i > 0)
        def _():
            out_ref[...] += part

    return pl.pallas_call(
        body,
        grid=(grid,),
        in_specs=[
            pl.BlockSpec((NC, bn, d), lambda i: (0, i, 0)),
            pl.BlockSpec((bn, d), lambda i: (i, 0)),
            pl.BlockSpec((bn, 1), lambda i: (i, 0)),
            pl.BlockSpec((1, d), lambda i: (0, 0)),
        ],
        out_specs=pl.BlockSpec((1, d), lambda i: (0, 0)),
        out_shape=jax.ShapeDtypeStruct((1, d), jnp.float32),
    )


def kernel(x, edge_index, edge_weight, W1, b1, W2, b2, W3, b3):
    n, d = x.shape
    e = edge_weight.shape[0]
    src, dst = edge_index[0], edge_index[1]

    chunk = NW * B
    nb = (e + chunk - 1) // chunk
    ep = nb * chunk
    pad = ep - e
    pt = nb * B
    srcp = jnp.concatenate([src, jnp.zeros((pad,), src.dtype)])
    dstp = jnp.concatenate([dst, jnp.zeros((pad,), dst.dtype)])
    ewp = jnp.concatenate([edge_weight, jnp.zeros((pad,), edge_weight.dtype)])
    src2 = srcp.reshape(NW * nb, B)
    dst2 = dstp.reshape(NW * nb, B)
    ew2 = ewp.reshape(NW * nb, B)

    bn = 2000
    n_pad = ((n + NS * 8 - 1) // (NS * 8)) * (NS * 8)
    deg_part = _build_deg_kernel(n, pt)(dstp, ewp)
    dis = _build_dis(n)(deg_part.reshape(NW, n))
    xs = _build_xs(n, d, bn)(x, dis, W1)

    spmm = _build_spmm_kernel(n_pad, d, nb)
    combine = _build_combine(n, d, bn)

    b1r = b1.reshape(1, d)
    b2r = b2.reshape(1, d)
    b3r = b3.reshape(1, d)

    acc = spmm(src2, dst2, ew2, xs)
    xs = combine(acc, xs, dis, b1r, W2)
    acc = spmm(src2, dst2, ew2, xs)
    xs = combine(acc, xs, dis, b2r, W3)
    acc = spmm(src2, dst2, ew2, xs)
    out = _build_final(n, d, bn)(acc, xs, dis, b3r)
    return out
